# baseline (speedup 1.0000x reference)
import functools

import jax
import jax.numpy as jnp
import numpy as np
from jax import lax
from jax.experimental import pallas as pl
from jax.experimental.pallas import tpu as pltpu
from jax.experimental.pallas import tpu_sc as plsc

NUM_CLASSES = 100000
DIM = 128
BATCH = 4096
NUM_SAMPLED = 1000
S_PAD = 1024

_NW = 32
_TRUE_PER_W = BATCH // _NW
_SAMP_PER_W = S_PAD // _NW


def _log_uniform_prob(ids_f):
    return (jnp.log(ids_f + 2.0) - jnp.log(ids_f + 1.0)) / jnp.log(
        float(NUM_CLASSES) + 1.0
    )


def _draw_sampled_ids():
    ks = jax.random.key(42)
    u = jax.random.uniform(ks, (NUM_SAMPLED,), dtype=jnp.float32)
    ids = jnp.floor(jnp.exp(u * jnp.log(float(NUM_CLASSES) + 1.0))) - 1.0
    return jnp.clip(ids, 0, NUM_CLASSES - 1).astype(jnp.int32)


def _lane_shuffle(x, idx):
    return lax.gather(
        x,
        idx[:, None],
        lax.GatherDimensionNumbers(
            offset_dims=(), collapsed_slice_dims=(0,), start_index_map=(0,)
        ),
        slice_sizes=(1,),
        mode=lax.GatherScatterMode.PROMISE_IN_BOUNDS,
    )


def _sc_gather_dot(table, idx, embed):
    mesh = plsc.VectorSubcoreMesh(core_axis_name="c", subcore_axis_name="s")
    per_w = _TRUE_PER_W

    @functools.partial(
        pl.kernel,
        out_type=jax.ShapeDtypeStruct((BATCH,), jnp.float32),
        mesh=mesh,
        scratch_types=(
            pltpu.VMEM((per_w,), jnp.int32),
            pltpu.VMEM((per_w, DIM), jnp.float32),
            pltpu.VMEM((per_w, DIM), jnp.float32),
            pltpu.VMEM((per_w,), jnp.float32),
            pltpu.SemaphoreType.DMA,
            pltpu.SemaphoreType.DMA,
        ),
    )
    def gather_dot_kernel(
        table_hbm, idx_hbm, emb_hbm, out_hbm,
        idx_v, rows_v, emb_v, dot_v, sem_g, sem_e,
    ):
        wid = lax.axis_index("s") * 2 + lax.axis_index("c")
        base = wid * per_w
        pltpu.sync_copy(idx_hbm.at[pl.ds(base, per_w)], idx_v)
        ce = pltpu.async_copy(emb_hbm.at[pl.ds(base, per_w)], emb_v, sem_e)
        cg = pltpu.async_copy(table_hbm.at[idx_v], rows_v, sem_g)
        ce.wait()
        cg.wait()
        lane = lax.iota(jnp.int32, 16)

        def group(g, _):
            gbase = pl.multiple_of(g * 16, 16)
            vals = jnp.zeros((16,), jnp.float32)
            for j in range(16):
                r = gbase + j
                acc = emb_v[r, pl.ds(0, 16)] * rows_v[r, pl.ds(0, 16)]
                for c in range(1, DIM // 16):
                    acc = acc + (
                        emb_v[r, pl.ds(16 * c, 16)] * rows_v[r, pl.ds(16 * c, 16)]
                    )
                t = acc + _lane_shuffle(acc, lane ^ 1)
                t = t + _lane_shuffle(t, lane ^ 2)
                t = t + _lane_shuffle(t, lane ^ 4)
                t = t + _lane_shuffle(t, lane ^ 8)
                vals = jnp.where(lane == j, t, vals)
            dot_v[pl.ds(gbase, 16)] = vals
            return _

        lax.fori_loop(0, per_w // 16, group, None)
        pltpu.sync_copy(dot_v, out_hbm.at[pl.ds(base, per_w)])

    return gather_dot_kernel(table, idx, embed)


def _sc_gather(table, idx, n_rows, per_w):
    mesh = plsc.VectorSubcoreMesh(core_axis_name="c", subcore_axis_name="s")

    @functools.partial(
        pl.kernel,
        out_type=jax.ShapeDtypeStruct((n_rows, DIM), jnp.float32),
        mesh=mesh,
        scratch_types=(
            pltpu.VMEM((per_w,), jnp.int32),
            pltpu.VMEM((per_w, DIM), jnp.float32),
            pltpu.SemaphoreType.DMA,
        ),
    )
    def gather_kernel(table_hbm, idx_hbm, out_hbm, idx_v, rows_v, sem):
        wid = lax.axis_index("s") * 2 + lax.axis_index("c")
        base = wid * per_w
        pltpu.sync_copy(idx_hbm.at[pl.ds(base, per_w)], idx_v)
        pltpu.async_copy(table_hbm.at[idx_v], rows_v, sem).wait()
        pltpu.sync_copy(rows_v, out_hbm.at[pl.ds(base, per_w)])

    return gather_kernel(table, idx)


_BLK = 2048


def _read_col(ref, i):
    full = ref[...]
    lane = lax.broadcasted_iota(jnp.int32, full.shape, 1)
    return jnp.sum(jnp.where(lane == i, full, 0), axis=1, keepdims=True)


def _write_col(ref, i, v):
    full = ref[...]
    lane = lax.broadcasted_iota(jnp.int32, full.shape, 1)
    ref[...] = jnp.where(lane == i, v, full)


_SHIFT = 32.0


def _tc_matmul_body(embed_ref, sampw_ref, lbl_ref, sid_ref, soff_ref, z_ref):
    i = pl.program_id(0)
    n_steps = BATCH // _BLK
    e = embed_ref[...].astype(jnp.bfloat16)
    sw = sampw_ref[...].astype(jnp.bfloat16)
    s = lax.dot_general(
        sw, e, (((1,), (1,)), ((), ())), preferred_element_type=jnp.float32
    )
    row_mask = lax.broadcasted_iota(jnp.int32, (n_steps, _BLK), 0) == i
    lbl_row = jnp.sum(
        jnp.where(row_mask, lbl_ref[...], 0), axis=0, keepdims=True
    )
    hit = lbl_row == sid_ref[...]
    ez = jnp.where(hit, 0.0, jnp.exp(s + soff_ref[...]))
    z = jnp.sum(ez, axis=0, keepdims=True)
    z_ref[...] = jnp.where(row_mask, z, z_ref[...])


def _tc_combine_body(tdot_ref, lbl_ref, z_ref, out_ref):
    lf = lbl_ref[...].astype(jnp.float32)
    true_expected = _log_uniform_prob(lf) * float(NUM_SAMPLED)
    t = tdot_ref[...] - jnp.log(true_expected)
    z = z_ref[...].reshape(BATCH)
    m = jnp.maximum(t, _SHIFT)
    lse = jnp.log(jnp.exp(t - m) + z * jnp.exp(_SHIFT - m)) + m
    out_ref[...] = lse - t


def kernel(softmax_weights, embed, label_idx, zero_bias):
    del zero_bias
    labels = label_idx.reshape(-1)
    lbl_rows = labels.reshape(BATCH // _BLK, _BLK)

    sampled_ids = _draw_sampled_ids()
    samp_idx_pad = jnp.concatenate(
        [sampled_ids, jnp.zeros((S_PAD - NUM_SAMPLED,), jnp.int32)]
    )
    sid_mask = jnp.concatenate(
        [sampled_ids, jnp.full((S_PAD - NUM_SAMPLED,), -1, jnp.int32)]
    ).reshape(S_PAD, 1)
    sampled_expected = _log_uniform_prob(
        sampled_ids.astype(jnp.float32)
    ) * float(NUM_SAMPLED)
    soff = jnp.concatenate(
        [-jnp.log(sampled_expected) - _SHIFT,
         jnp.full((S_PAD - NUM_SAMPLED,), -1e30, jnp.float32)]
    ).reshape(S_PAD, 1)

    samp_w = _sc_gather(softmax_weights, samp_idx_pad, S_PAD, _SAMP_PER_W)
    t_dot = _sc_gather_dot(softmax_weights, labels, embed)

    n_steps = BATCH // _BLK
    grid = (n_steps,)
    z_s = pl.pallas_call(
        _tc_matmul_body,
        grid=grid,
        in_specs=[
            pl.BlockSpec((_BLK, DIM), lambda i: (i, 0)),
            pl.BlockSpec((S_PAD, DIM), lambda i: (0, 0)),
            pl.BlockSpec((BATCH // _BLK, _BLK), lambda i: (0, 0)),
            pl.BlockSpec((S_PAD, 1), lambda i: (0, 0)),
            pl.BlockSpec((S_PAD, 1), lambda i: (0, 0)),
        ],
        out_specs=pl.BlockSpec((BATCH // _BLK, _BLK), lambda i: (0, 0)),
        out_shape=jax.ShapeDtypeStruct((n_steps, _BLK), jnp.float32),
    )(embed, samp_w, lbl_rows, sid_mask, soff)


    loss = pl.pallas_call(
        _tc_combine_body,
        out_shape=jax.ShapeDtypeStruct((BATCH,), jnp.float32),
    )(t_dot, labels, z_s)

    return loss


# speedup vs baseline: 1.0035x; 1.0035x over previous
import functools

import jax
import jax.numpy as jnp
from jax import lax
from jax.experimental import pallas as pl
from jax.experimental.pallas import tpu as pltpu
from jax.experimental.pallas import tpu_sc as plsc

NUM_CLASSES = 100000
DIM = 128
BATCH = 4096
NUM_SAMPLED = 1000
S_PAD = 1024

_NW = 32
_TRUE_PER_W = BATCH // _NW
_SAMP_PER_W = S_PAD // _NW


def _log_uniform_prob(ids_f):
    return (jnp.log(ids_f + 2.0) - jnp.log(ids_f + 1.0)) / jnp.log(
        float(NUM_CLASSES) + 1.0
    )


def _draw_sampled_ids():
    ks = jax.random.key(42)
    u = jax.random.uniform(ks, (NUM_SAMPLED,), dtype=jnp.float32)
    ids = jnp.floor(jnp.exp(u * jnp.log(float(NUM_CLASSES) + 1.0))) - 1.0
    return jnp.clip(ids, 0, NUM_CLASSES - 1).astype(jnp.int32)


def _lane_shuffle(x, idx):
    return lax.gather(
        x,
        idx[:, None],
        lax.GatherDimensionNumbers(
            offset_dims=(), collapsed_slice_dims=(0,), start_index_map=(0,)
        ),
        slice_sizes=(1,),
        mode=lax.GatherScatterMode.PROMISE_IN_BOUNDS,
    )


def _sc_gather_dot(table, idx, embed):
    mesh = plsc.VectorSubcoreMesh(core_axis_name="c", subcore_axis_name="s")
    per_w = _TRUE_PER_W

    @functools.partial(
        pl.kernel,
        out_type=jax.ShapeDtypeStruct((BATCH,), jnp.float32),
        mesh=mesh,
        scratch_types=(
            pltpu.VMEM((per_w,), jnp.int32),
            pltpu.VMEM((per_w, DIM), jnp.float32),
            pltpu.VMEM((per_w, DIM), jnp.float32),
            pltpu.VMEM((per_w,), jnp.float32),
            pltpu.SemaphoreType.DMA,
            pltpu.SemaphoreType.DMA,
        ),
    )
    def gather_dot_kernel(
        table_hbm, idx_hbm, emb_hbm, out_hbm,
        idx_v, rows_v, emb_v, dot_v, sem_g, sem_e,
    ):
        wid = lax.axis_index("s") * 2 + lax.axis_index("c")
        base = wid * per_w
        pltpu.sync_copy(idx_hbm.at[pl.ds(base, per_w)], idx_v)
        ce = pltpu.async_copy(emb_hbm.at[pl.ds(base, per_w)], emb_v, sem_e)
        cg = pltpu.async_copy(table_hbm.at[idx_v], rows_v, sem_g)
        ce.wait()
        cg.wait()
        lane = lax.iota(jnp.int32, 16)

        def group(g, _):
            gbase = pl.multiple_of(g * 16, 16)
            vals = jnp.zeros((16,), jnp.float32)
            for j in range(16):
                r = gbase + j
                acc = emb_v[r, pl.ds(0, 16)] * rows_v[r, pl.ds(0, 16)]
                for c in range(1, DIM // 16):
                    acc = acc + (
                        emb_v[r, pl.ds(16 * c, 16)] * rows_v[r, pl.ds(16 * c, 16)]
                    )
                t = acc + _lane_shuffle(acc, lane ^ 1)
                t = t + _lane_shuffle(t, lane ^ 2)
                t = t + _lane_shuffle(t, lane ^ 4)
                t = t + _lane_shuffle(t, lane ^ 8)
                vals = jnp.where(lane == j, t, vals)
            dot_v[pl.ds(gbase, 16)] = vals
            return _

        lax.fori_loop(0, per_w // 16, group, None)
        pltpu.sync_copy(dot_v, out_hbm.at[pl.ds(base, per_w)])

    return gather_dot_kernel(table, idx, embed)


def _sc_gather(table, idx, n_rows, per_w):
    mesh = plsc.VectorSubcoreMesh(core_axis_name="c", subcore_axis_name="s")

    @functools.partial(
        pl.kernel,
        out_type=jax.ShapeDtypeStruct((n_rows, DIM), jnp.float32),
        mesh=mesh,
        scratch_types=(
            pltpu.VMEM((per_w,), jnp.int32),
            pltpu.VMEM((per_w, DIM), jnp.float32),
            pltpu.SemaphoreType.DMA,
        ),
    )
    def gather_kernel(table_hbm, idx_hbm, out_hbm, idx_v, rows_v, sem):
        wid = lax.axis_index("s") * 2 + lax.axis_index("c")
        base = wid * per_w
        pltpu.sync_copy(idx_hbm.at[pl.ds(base, per_w)], idx_v)
        pltpu.async_copy(table_hbm.at[idx_v], rows_v, sem).wait()
        pltpu.sync_copy(rows_v, out_hbm.at[pl.ds(base, per_w)])

    return gather_kernel(table, idx)


_BLK = 1024


_SHIFT = 32.0


def _tc_matmul_body(embed_ref, sampw_ref, lbl_ref, sid_ref, soff_ref, z_ref):
    i = pl.program_id(0)
    n_steps = BATCH // _BLK
    e = embed_ref[...].astype(jnp.bfloat16)
    sw = sampw_ref[...].astype(jnp.bfloat16)
    s = lax.dot_general(
        sw, e, (((1,), (1,)), ((), ())), preferred_element_type=jnp.float32
    )
    row_mask = lax.broadcasted_iota(jnp.int32, (n_steps, _BLK), 0) == i
    lbl_row = jnp.sum(
        jnp.where(row_mask, lbl_ref[...], 0), axis=0, keepdims=True
    )
    hit = lbl_row == sid_ref[...]
    ez = jnp.where(hit, 0.0, jnp.exp(s + soff_ref[...]))
    z = jnp.sum(ez, axis=0, keepdims=True)
    z_ref[...] = jnp.where(row_mask, z, z_ref[...])


def _tc_combine_body(tdot_ref, lbl_ref, z_ref, out_ref):
    lf = lbl_ref[...].astype(jnp.float32)
    true_expected = _log_uniform_prob(lf) * float(NUM_SAMPLED)
    t = tdot_ref[...] - jnp.log(true_expected)
    z = z_ref[...].reshape(BATCH)
    m = jnp.maximum(t, _SHIFT)
    lse = jnp.log(jnp.exp(t - m) + z * jnp.exp(_SHIFT - m)) + m
    out_ref[...] = lse - t


def kernel(softmax_weights, embed, label_idx, zero_bias):
    del zero_bias
    labels = label_idx.reshape(-1)
    lbl_rows = labels.reshape(BATCH // _BLK, _BLK)

    sampled_ids = _draw_sampled_ids()
    samp_idx_pad = jnp.concatenate(
        [sampled_ids, jnp.zeros((S_PAD - NUM_SAMPLED,), jnp.int32)]
    )
    sid_mask = jnp.concatenate(
        [sampled_ids, jnp.full((S_PAD - NUM_SAMPLED,), -1, jnp.int32)]
    ).reshape(S_PAD, 1)
    sampled_expected = _log_uniform_prob(
        sampled_ids.astype(jnp.float32)
    ) * float(NUM_SAMPLED)
    soff = jnp.concatenate(
        [-jnp.log(sampled_expected) - _SHIFT,
         jnp.full((S_PAD - NUM_SAMPLED,), -1e30, jnp.float32)]
    ).reshape(S_PAD, 1)

    samp_w = _sc_gather(softmax_weights, samp_idx_pad, S_PAD, _SAMP_PER_W)
    t_dot = _sc_gather_dot(softmax_weights, labels, embed)

    n_steps = BATCH // _BLK
    grid = (n_steps,)
    z_s = pl.pallas_call(
        _tc_matmul_body,
        grid=grid,
        in_specs=[
            pl.BlockSpec((_BLK, DIM), lambda i: (i, 0)),
            pl.BlockSpec((S_PAD, DIM), lambda i: (0, 0)),
            pl.BlockSpec((BATCH // _BLK, _BLK), lambda i: (0, 0)),
            pl.BlockSpec((S_PAD, 1), lambda i: (0, 0)),
            pl.BlockSpec((S_PAD, 1), lambda i: (0, 0)),
        ],
        out_specs=pl.BlockSpec((BATCH // _BLK, _BLK), lambda i: (0, 0)),
        out_shape=jax.ShapeDtypeStruct((n_steps, _BLK), jnp.float32),
    )(embed, samp_w, lbl_rows, sid_mask, soff)


    loss = pl.pallas_call(
        _tc_combine_body,
        out_shape=jax.ShapeDtypeStruct((BATCH,), jnp.float32),
    )(t_dot, labels, z_s)

    return loss


# speedup vs baseline: 1.0205x; 1.0170x over previous
import functools

import jax
import jax.numpy as jnp
from jax import lax
from jax.experimental import pallas as pl
from jax.experimental.pallas import tpu as pltpu
from jax.experimental.pallas import tpu_sc as plsc

NUM_CLASSES = 100000
DIM = 128
BATCH = 4096
NUM_SAMPLED = 1000
S_PAD = 1024

_NW = 32
_TRUE_PER_W = BATCH // _NW
_SAMP_PER_W = S_PAD // _NW


def _log_uniform_prob(ids_f):
    return (jnp.log(ids_f + 2.0) - jnp.log(ids_f + 1.0)) / jnp.log(
        float(NUM_CLASSES) + 1.0
    )


def _draw_sampled_ids():
    ks = jax.random.key(42)
    u = jax.random.uniform(ks, (NUM_SAMPLED,), dtype=jnp.float32)
    ids = jnp.floor(jnp.exp(u * jnp.log(float(NUM_CLASSES) + 1.0))) - 1.0
    return jnp.clip(ids, 0, NUM_CLASSES - 1).astype(jnp.int32)


def _lane_shuffle(x, idx):
    return lax.gather(
        x,
        idx[:, None],
        lax.GatherDimensionNumbers(
            offset_dims=(), collapsed_slice_dims=(0,), start_index_map=(0,)
        ),
        slice_sizes=(1,),
        mode=lax.GatherScatterMode.PROMISE_IN_BOUNDS,
    )


def _sc_gather_dot(table, idx, embed):
    mesh = plsc.VectorSubcoreMesh(core_axis_name="c", subcore_axis_name="s")
    per_w = _TRUE_PER_W

    @functools.partial(
        pl.kernel,
        out_type=jax.ShapeDtypeStruct((BATCH,), jnp.float32),
        mesh=mesh,
        scratch_types=(
            pltpu.VMEM((per_w,), jnp.int32),
            pltpu.VMEM((per_w, DIM), jnp.float32),
            pltpu.VMEM((per_w, DIM), jnp.float32),
            pltpu.VMEM((per_w,), jnp.float32),
            pltpu.SemaphoreType.DMA,
            pltpu.SemaphoreType.DMA,
        ),
    )
    def gather_dot_kernel(
        table_hbm, idx_hbm, emb_hbm, out_hbm,
        idx_v, rows_v, emb_v, dot_v, sem_g, sem_e,
    ):
        wid = lax.axis_index("s") * 2 + lax.axis_index("c")
        base = wid * per_w
        ce = pltpu.async_copy(emb_hbm.at[pl.ds(base, per_w)], emb_v, sem_e)
        pltpu.sync_copy(idx_hbm.at[pl.ds(base, per_w)], idx_v)
        cg = pltpu.async_copy(table_hbm.at[idx_v], rows_v, sem_g)
        ce.wait()
        cg.wait()
        lane = lax.iota(jnp.int32, 16)

        def group(g, _):
            gbase = pl.multiple_of(g * 16, 16)
            vals = jnp.zeros((16,), jnp.float32)
            for j in range(16):
                r = gbase + j
                acc = emb_v[r, pl.ds(0, 16)] * rows_v[r, pl.ds(0, 16)]
                for c in range(1, DIM // 16):
                    acc = acc + (
                        emb_v[r, pl.ds(16 * c, 16)] * rows_v[r, pl.ds(16 * c, 16)]
                    )
                t = acc + _lane_shuffle(acc, lane ^ 1)
                t = t + _lane_shuffle(t, lane ^ 2)
                t = t + _lane_shuffle(t, lane ^ 4)
                t = t + _lane_shuffle(t, lane ^ 8)
                vals = jnp.where(lane == j, t, vals)
            dot_v[pl.ds(gbase, 16)] = vals
            return _

        lax.fori_loop(0, per_w // 16, group, None)
        pltpu.sync_copy(dot_v, out_hbm.at[pl.ds(base, per_w)])

    return gather_dot_kernel(table, idx, embed)


def _sc_gather(table, idx, n_rows, per_w):
    mesh = plsc.VectorSubcoreMesh(core_axis_name="c", subcore_axis_name="s")

    @functools.partial(
        pl.kernel,
        out_type=jax.ShapeDtypeStruct((n_rows, DIM), jnp.float32),
        mesh=mesh,
        scratch_types=(
            pltpu.VMEM((per_w,), jnp.int32),
            pltpu.VMEM((per_w, DIM), jnp.float32),
            pltpu.SemaphoreType.DMA,
        ),
    )
    def gather_kernel(table_hbm, idx_hbm, out_hbm, idx_v, rows_v, sem):
        wid = lax.axis_index("s") * 2 + lax.axis_index("c")
        base = wid * per_w
        pltpu.sync_copy(idx_hbm.at[pl.ds(base, per_w)], idx_v)
        pltpu.async_copy(table_hbm.at[idx_v], rows_v, sem).wait()
        pltpu.sync_copy(rows_v, out_hbm.at[pl.ds(base, per_w)])

    return gather_kernel(table, idx)


_BLK = 1024


_SHIFT = 32.0


def _tc_matmul_body(embed_ref, sampw_ref, lbl_ref, sid_ref, soff_ref, z_ref):
    i = pl.program_id(0)
    n_steps = BATCH // _BLK
    e = embed_ref[...].astype(jnp.bfloat16)
    sw = sampw_ref[...].astype(jnp.bfloat16)
    s = lax.dot_general(
        sw, e, (((1,), (1,)), ((), ())), preferred_element_type=jnp.float32
    )
    row_mask = lax.broadcasted_iota(jnp.int32, (n_steps, _BLK), 0) == i
    lbl_row = jnp.sum(
        jnp.where(row_mask, lbl_ref[...], 0), axis=0, keepdims=True
    )
    hit = lbl_row == sid_ref[...]
    ez = jnp.where(hit, 0.0, jnp.exp(s + soff_ref[...]))
    z = jnp.sum(ez, axis=0, keepdims=True)
    z_ref[...] = jnp.where(row_mask, z, z_ref[...])


def _tc_combine_body(tdot_ref, lbl_ref, z_ref, out_ref):
    lf = lbl_ref[...].astype(jnp.float32)
    true_expected = _log_uniform_prob(lf) * float(NUM_SAMPLED)
    t = tdot_ref[...] - jnp.log(true_expected)
    z = z_ref[...].reshape(BATCH)
    m = jnp.maximum(t, _SHIFT)
    lse = jnp.log(jnp.exp(t - m) + z * jnp.exp(_SHIFT - m)) + m
    out_ref[...] = lse - t


def kernel(softmax_weights, embed, label_idx, zero_bias):
    del zero_bias
    labels = label_idx.reshape(-1)
    lbl_rows = labels.reshape(BATCH // _BLK, _BLK)

    sampled_ids = _draw_sampled_ids()
    samp_idx_pad = jnp.concatenate(
        [sampled_ids, jnp.zeros((S_PAD - NUM_SAMPLED,), jnp.int32)]
    )
    sid_mask = jnp.concatenate(
        [sampled_ids, jnp.full((S_PAD - NUM_SAMPLED,), -1, jnp.int32)]
    ).reshape(S_PAD, 1)
    sampled_expected = _log_uniform_prob(
        sampled_ids.astype(jnp.float32)
    ) * float(NUM_SAMPLED)
    soff = jnp.concatenate(
        [-jnp.log(sampled_expected) - _SHIFT,
         jnp.full((S_PAD - NUM_SAMPLED,), -1e30, jnp.float32)]
    ).reshape(S_PAD, 1)

    samp_w = _sc_gather(softmax_weights, samp_idx_pad, S_PAD, _SAMP_PER_W)
    t_dot = _sc_gather_dot(softmax_weights, labels, embed)

    n_steps = BATCH // _BLK
    grid = (n_steps,)
    z_s = pl.pallas_call(
        _tc_matmul_body,
        grid=grid,
        in_specs=[
            pl.BlockSpec((_BLK, DIM), lambda i: (i, 0)),
            pl.BlockSpec((S_PAD, DIM), lambda i: (0, 0)),
            pl.BlockSpec((BATCH // _BLK, _BLK), lambda i: (0, 0)),
            pl.BlockSpec((S_PAD, 1), lambda i: (0, 0)),
            pl.BlockSpec((S_PAD, 1), lambda i: (0, 0)),
        ],
        out_specs=pl.BlockSpec((BATCH // _BLK, _BLK), lambda i: (0, 0)),
        out_shape=jax.ShapeDtypeStruct((n_steps, _BLK), jnp.float32),
    )(embed, samp_w, lbl_rows, sid_mask, soff)


    loss = pl.pallas_call(
        _tc_combine_body,
        out_shape=jax.ShapeDtypeStruct((BATCH,), jnp.float32),
    )(t_dot, labels, z_s)

    return loss
